# baseline (speedup 1.0000x reference)
import functools

import jax
import jax.numpy as jnp
from jax import lax
from jax.experimental import pallas as pl
from jax.experimental.pallas import tpu as pltpu
from jax.experimental.pallas import tpu_sc as plsc

D_MODEL = 1024
SCALE = 32.0
NW = 32
LANES = 16


@functools.cache
def _make_kernel(B, S, CH):
    T = B * S
    tok_per_w = T // NW
    n_ch = tok_per_w // CH
    mesh = plsc.VectorSubcoreMesh(core_axis_name="c", subcore_axis_name="s")

    @functools.partial(
        pl.kernel,
        mesh=mesh,
        out_type=jax.ShapeDtypeStruct((T, D_MODEL), jnp.float32),
        scratch_types=[
            pltpu.VMEM((tok_per_w,), jnp.int32),
            pltpu.VMEM((CH, D_MODEL), jnp.float32),
            pltpu.VMEM((CH, D_MODEL), jnp.float32),
            pltpu.SemaphoreType.DMA,
        ],
    )
    def emb_kernel(x_hbm, table_hbm, pos_hbm, out_hbm, idx_v, rows_v, pos_v, sem):
        wid = lax.axis_index("s") * 2 + lax.axis_index("c")
        base = wid * tok_per_w
        s_start = base % S
        pltpu.sync_copy(x_hbm.at[pl.ds(base, tok_per_w)], idx_v)

        def chunk_body(ch, _):
            off = ch * CH
            gather = pltpu.async_copy(
                table_hbm.at[idx_v.at[pl.ds(off, CH)]], rows_v, sem)
            pltpu.sync_copy(pos_hbm.at[pl.ds(s_start + off, CH)], pos_v)
            gather.wait()

            def row_body(r, _):
                def vec_body(j, _):
                    o = j * LANES
                    rows_v[r, pl.ds(o, LANES)] = (
                        rows_v[r, pl.ds(o, LANES)] * SCALE
                        + pos_v[r, pl.ds(o, LANES)])
                    return 0
                return lax.fori_loop(0, D_MODEL // LANES, vec_body, 0)

            lax.fori_loop(0, CH, row_body, 0)
            pltpu.sync_copy(rows_v, out_hbm.at[pl.ds(base + off, CH)])
            return 0

        lax.fori_loop(0, n_ch, chunk_body, 0)

    return emb_kernel


def kernel(x, emb_table, pos_enc):
    B, S = x.shape
    out = _make_kernel(B, S, 32)(x.reshape(-1), emb_table, pos_enc)
    return out.reshape(B, S, D_MODEL)


# speedup vs baseline: 1.8555x; 1.8555x over previous
import functools
import math

import jax
import jax.numpy as jnp
from jax import lax
from jax.experimental import pallas as pl
from jax.experimental.pallas import tpu as pltpu
from jax.experimental.pallas import tpu_sc as plsc

NW = 32
LANES = 16
CH = 16


@functools.cache
def _make_kernel(B, S, D):
    scale = math.sqrt(D)
    tok_w = S // NW
    half = tok_w // 2
    n_ch = (B * tok_w) // CH
    per_half = n_ch // 2
    mesh = plsc.VectorSubcoreMesh(core_axis_name="c", subcore_axis_name="s")

    @functools.partial(
        pl.kernel,
        mesh=mesh,
        out_type=jax.ShapeDtypeStruct((B * S, D), jnp.float32),
        scratch_types=[
            pltpu.VMEM((B * tok_w,), jnp.int32),
            pltpu.VMEM((2, CH, D), jnp.float32),
            pltpu.VMEM((2, CH, D), jnp.float32),
            pltpu.VMEM((half, D), jnp.float32),
            pltpu.SemaphoreType.DMA,
            pltpu.SemaphoreType.DMA,
            pltpu.SemaphoreType.DMA,
            pltpu.SemaphoreType.DMA,
        ],
    )
    def emb_kernel(x_hbm, table_hbm, pos_hbm, out_hbm,
                   idx_v, rows_v, stage_v, pos_v, gs0, gs1, os0, os1):
        wid = lax.axis_index("s") * 2 + lax.axis_index("c")
        sbase = wid * tok_w
        gsems = (gs0, gs1)
        osems = (os0, os1)

        for b in range(B):
            pltpu.sync_copy(x_hbm.at[pl.ds(b * S + sbase, tok_w)],
                            idx_v.at[pl.ds(b * tok_w, tok_w)])
        pltpu.sync_copy(pos_hbm.at[pl.ds(sbase, half)], pos_v)

        def params(ch):
            h, r = divmod(ch, per_half)
            b, c2 = divmod(r, half // CH)
            toff = h * half + c2 * CH
            return h, b, toff

        def start_gather(ch):
            _, b, toff = params(ch)
            s = ch % 2
            return pltpu.async_copy(
                table_hbm.at[idx_v.at[pl.ds(b * tok_w + toff, CH)]],
                rows_v.at[s], gsems[s])

        g = {0: start_gather(0), 1: start_gather(1)}
        o = {}
        for ch in range(n_ch):
            s = ch % 2
            h, b, toff = params(ch)
            if ch == per_half:
                pltpu.sync_copy(pos_hbm.at[pl.ds(sbase + half, half)], pos_v)
            g[ch].wait()
            if ch >= 2:
                o[ch - 2].wait()
            po = toff - h * half

            def row_body(r, _, s=s, po=po):
                for j in range(D // LANES):
                    off = j * LANES
                    stage_v[s, r, pl.ds(off, LANES)] = (
                        rows_v[s, r, pl.ds(off, LANES)] * scale
                        + pos_v[po + r, pl.ds(off, LANES)])
                return 0

            lax.fori_loop(0, CH, row_body, 0)
            if ch + 2 < n_ch:
                g[ch + 2] = start_gather(ch + 2)
            o[ch] = pltpu.async_copy(
                stage_v.at[s],
                out_hbm.at[pl.ds(b * S + sbase + toff, CH)], osems[s])
        o[n_ch - 2].wait()
        o[n_ch - 1].wait()

    return emb_kernel


def kernel(x, emb_table, pos_enc):
    B, S = x.shape
    D = emb_table.shape[1]
    out = _make_kernel(B, S, D)(x.reshape(-1), emb_table, pos_enc)
    return out.reshape(B, S, D)
